# baseline (speedup 1.0000x reference)
import jax
import jax.numpy as jnp
from jax.experimental import pallas as pl
from jax.experimental.pallas import tpu as pltpu

_N = 32768
_C = 64
_K = 4096
_KR = 4096
_BN = 256


def _exact_gather(table, idx):
    n, k = idx.shape[0], table.shape[0]
    iota = jax.lax.broadcasted_iota(jnp.int32, (n, k), 1)
    onehot = (iota == idx[:, None]).astype(jnp.bfloat16)
    tu = jax.lax.bitcast_convert_type(table, jnp.int32)
    acc = jnp.zeros((n, table.shape[1]), jnp.int32)
    for shift in (0, 8, 16, 24):
        plane = ((tu >> shift) & 0xFF).astype(jnp.bfloat16)
        g = jax.lax.dot_general(onehot, plane, (((1,), (0,)), ((), ())),
                                preferred_element_type=jnp.float32)
        acc = acc | (g.astype(jnp.int32) << shift)
    return jax.lax.bitcast_convert_type(acc, jnp.float32)


def _argmin_rows(d, k):
    iota = jax.lax.broadcasted_iota(jnp.int32, d.shape, 1)
    mind = jnp.min(d, axis=1, keepdims=True)
    return jnp.min(jnp.where(d == mind, iota, k), axis=1)


def _rvq_body(x_ref, cb_ref, rcb_ref, cbsq_ref, rcbsq_ref,
              quant_ref, idx_ref, quantr_ref, idxr_ref):
    x = x_ref[...]
    cb = cb_ref[...]
    rcb = rcb_ref[...]

    x_sq = jnp.sum(x * x, axis=1, keepdims=True)
    xc = jax.lax.dot_general(x, cb, (((1,), (1,)), ((), ())),
                             preferred_element_type=jnp.float32)
    d = x_sq - 2.0 * xc + cbsq_ref[...]
    idx = _argmin_rows(d, _K)

    quant = _exact_gather(cb, idx)

    r = x - quant
    r_sq = jnp.sum(r * r, axis=1, keepdims=True)
    rc = jax.lax.dot_general(r, rcb, (((1,), (1,)), ((), ())),
                             preferred_element_type=jnp.float32)
    d2 = r_sq - 2.0 * rc + rcbsq_ref[...]
    idx2 = _argmin_rows(d2, _KR)

    quant2 = _exact_gather(rcb, idx2)

    quant_ref[...] = quant
    idx_ref[...] = idx[:, None]
    quantr_ref[...] = quant2
    idxr_ref[...] = idx2[:, None]


def kernel(x, codebook, residual_codebook):
    cbsq = jnp.sum(codebook * codebook, axis=1)[None, :]
    rcbsq = jnp.sum(residual_codebook * residual_codebook, axis=1)[None, :]

    grid = (_N // _BN,)
    out = pl.pallas_call(
        _rvq_body,
        grid=grid,
        in_specs=[
            pl.BlockSpec((_BN, _C), lambda i: (i, 0)),
            pl.BlockSpec((_K, _C), lambda i: (0, 0)),
            pl.BlockSpec((_KR, _C), lambda i: (0, 0)),
            pl.BlockSpec((1, _K), lambda i: (0, 0)),
            pl.BlockSpec((1, _KR), lambda i: (0, 0)),
        ],
        out_specs=[
            pl.BlockSpec((_BN, _C), lambda i: (i, 0)),
            pl.BlockSpec((_BN, 1), lambda i: (i, 0)),
            pl.BlockSpec((_BN, _C), lambda i: (i, 0)),
            pl.BlockSpec((_BN, 1), lambda i: (i, 0)),
        ],
        out_shape=[
            jax.ShapeDtypeStruct((_N, _C), jnp.float32),
            jax.ShapeDtypeStruct((_N, 1), jnp.int32),
            jax.ShapeDtypeStruct((_N, _C), jnp.float32),
            jax.ShapeDtypeStruct((_N, 1), jnp.int32),
        ],
    )(x, codebook, residual_codebook, cbsq, rcbsq)
    quant, idx, quant_r, idx_r = out
    return (quant, idx[:, 0], quant_r, idx_r[:, 0])


# speedup vs baseline: 1.5371x; 1.5371x over previous
import jax
import jax.numpy as jnp
from jax.experimental import pallas as pl
from jax.experimental.pallas import tpu as pltpu

_N = 32768
_C = 64
_K = 4096
_KR = 4096
_BN = 512


def _byte_planes(table):
    tu = jax.lax.bitcast_convert_type(table, jnp.int32)
    planes = [((tu >> s) & 0xFF).astype(jnp.bfloat16) for s in (0, 8, 16, 24)]
    return jnp.concatenate(planes, axis=1)


def _exact_gather(planes, idx, k):
    n = idx.shape[0]
    iota = jax.lax.broadcasted_iota(jnp.int32, (n, k), 1)
    onehot = (iota == idx[:, None]).astype(jnp.bfloat16)
    g = jax.lax.dot_general(onehot, planes, (((1,), (0,)), ((), ())),
                            preferred_element_type=jnp.float32)
    acc = g[:, 0:_C].astype(jnp.int32)
    acc = acc | (g[:, _C:2 * _C].astype(jnp.int32) << 8)
    acc = acc | (g[:, 2 * _C:3 * _C].astype(jnp.int32) << 16)
    acc = acc | (g[:, 3 * _C:4 * _C].astype(jnp.int32) << 24)
    return jax.lax.bitcast_convert_type(acc, jnp.float32)


_W = 128


def _argmin_scan(x_sq, xc2, csq, k):
    n = x_sq.shape[0]
    best = jnp.full((n, _W), jnp.inf, jnp.float32)
    bests = jnp.zeros((n, _W), jnp.int32)
    for s in range(k // _W):
        d_s = (x_sq + xc2[:, s * _W:(s + 1) * _W]) + csq[:, s * _W:(s + 1) * _W]
        cond = d_s < best
        best = jnp.where(cond, d_s, best)
        bests = jnp.where(cond, s, bests)
    mind = jnp.min(best, axis=1, keepdims=True)
    lane = jax.lax.broadcasted_iota(jnp.int32, (n, _W), 1)
    cand = jnp.where(best == mind, bests * _W + lane, k)
    return jnp.min(cand, axis=1)


def _rvq_body(x_ref, cbm2_ref, rcbm2_ref, cbsq_ref, rcbsq_ref,
              cbpl_ref, rcbpl_ref,
              quant_ref, idx_ref, quantr_ref, idxr_ref):
    x = x_ref[...]

    x_sq = jnp.sum(x * x, axis=1, keepdims=True)
    xc2 = jax.lax.dot_general(x, cbm2_ref[...], (((1,), (1,)), ((), ())),
                              preferred_element_type=jnp.float32)
    idx = _argmin_scan(x_sq, xc2, cbsq_ref[...], _K)
    quant = _exact_gather(cbpl_ref[...], idx, _K)

    r = x - quant
    r_sq = jnp.sum(r * r, axis=1, keepdims=True)
    rc2 = jax.lax.dot_general(r, rcbm2_ref[...], (((1,), (1,)), ((), ())),
                              preferred_element_type=jnp.float32)
    idx2 = _argmin_scan(r_sq, rc2, rcbsq_ref[...], _KR)
    quant2 = _exact_gather(rcbpl_ref[...], idx2, _KR)

    quant_ref[...] = quant
    idx_ref[...] = idx[:, None]
    quantr_ref[...] = quant2
    idxr_ref[...] = idx2[:, None]


def kernel(x, codebook, residual_codebook):
    cbm2 = -2.0 * codebook
    rcbm2 = -2.0 * residual_codebook
    cbsq = jnp.sum(codebook * codebook, axis=1)[None, :]
    rcbsq = jnp.sum(residual_codebook * residual_codebook, axis=1)[None, :]
    cbpl = _byte_planes(codebook)
    rcbpl = _byte_planes(residual_codebook)

    grid = (_N // _BN,)
    out = pl.pallas_call(
        _rvq_body,
        grid=grid,
        in_specs=[
            pl.BlockSpec((_BN, _C), lambda i: (i, 0)),
            pl.BlockSpec((_K, _C), lambda i: (0, 0)),
            pl.BlockSpec((_KR, _C), lambda i: (0, 0)),
            pl.BlockSpec((1, _K), lambda i: (0, 0)),
            pl.BlockSpec((1, _KR), lambda i: (0, 0)),
            pl.BlockSpec((_K, 4 * _C), lambda i: (0, 0)),
            pl.BlockSpec((_KR, 4 * _C), lambda i: (0, 0)),
        ],
        out_specs=[
            pl.BlockSpec((_BN, _C), lambda i: (i, 0)),
            pl.BlockSpec((_BN, 1), lambda i: (i, 0)),
            pl.BlockSpec((_BN, _C), lambda i: (i, 0)),
            pl.BlockSpec((_BN, 1), lambda i: (i, 0)),
        ],
        out_shape=[
            jax.ShapeDtypeStruct((_N, _C), jnp.float32),
            jax.ShapeDtypeStruct((_N, 1), jnp.int32),
            jax.ShapeDtypeStruct((_N, _C), jnp.float32),
            jax.ShapeDtypeStruct((_N, 1), jnp.int32),
        ],
    )(x, cbm2, rcbm2, cbsq, rcbsq, cbpl, rcbpl)
    quant, idx, quant_r, idx_r = out
    return (quant, idx[:, 0], quant_r, idx_r[:, 0])


# speedup vs baseline: 1.8821x; 1.2245x over previous
import jax
import jax.numpy as jnp
from jax.experimental import pallas as pl
from jax.experimental.pallas import tpu as pltpu

_N = 32768
_C = 64
_K = 4096
_KR = 4096
_BN = 512


def _byte_planes(table):
    tu = jax.lax.bitcast_convert_type(table, jnp.int32)
    planes = [(((tu >> s) & 0xFF) - 128).astype(jnp.int8)
              for s in (0, 8, 16, 24)]
    return jnp.concatenate(planes, axis=1)


def _exact_gather(planes, idx, k):
    n = idx.shape[0]
    iota = jax.lax.broadcasted_iota(jnp.int32, (n, k), 1)
    onehot = (iota == idx[:, None]).astype(jnp.int8)
    g = jax.lax.dot_general(onehot, planes, (((1,), (0,)), ((), ())),
                            preferred_element_type=jnp.int32) + 128
    acc = g[:, 0:_C]
    acc = acc | (g[:, _C:2 * _C] << 8)
    acc = acc | (g[:, 2 * _C:3 * _C] << 16)
    acc = acc | (g[:, 3 * _C:4 * _C] << 24)
    return jax.lax.bitcast_convert_type(acc, jnp.float32)


_W = 128


def _argmin_scan(x_sq, xc2, csq, k):
    n = x_sq.shape[0]
    best = jnp.full((n, _W), jnp.inf, jnp.float32)
    bests = jnp.zeros((n, _W), jnp.int32)
    for s in range(k // _W):
        d_s = (x_sq + xc2[:, s * _W:(s + 1) * _W]) + csq[:, s * _W:(s + 1) * _W]
        cond = d_s < best
        best = jnp.where(cond, d_s, best)
        bests = jnp.where(cond, s, bests)
    mind = jnp.min(best, axis=1, keepdims=True)
    lane = jax.lax.broadcasted_iota(jnp.int32, (n, _W), 1)
    cand = jnp.where(best == mind, bests * _W + lane, k)
    return jnp.min(cand, axis=1)


def _rvq_body(x_ref, cbm2_ref, rcbm2_ref, cbsq_ref, rcbsq_ref,
              cbpl_ref, rcbpl_ref,
              quant_ref, idx_ref, quantr_ref, idxr_ref,
              r_buf):
    i = pl.program_id(0)

    r = r_buf[(i + 1) % 2]
    r_sq = jnp.sum(r * r, axis=1, keepdims=True)
    rc2 = jax.lax.dot_general(r, rcbm2_ref[...], (((1,), (1,)), ((), ())),
                              preferred_element_type=jnp.float32)
    idx2 = _argmin_scan(r_sq, rc2, rcbsq_ref[...], _KR)
    quant2 = _exact_gather(rcbpl_ref[...], idx2, _KR)
    quantr_ref[...] = quant2
    idxr_ref[...] = idx2[:, None]

    x = x_ref[...]
    x_sq = jnp.sum(x * x, axis=1, keepdims=True)
    xc2 = jax.lax.dot_general(x, cbm2_ref[...], (((1,), (1,)), ((), ())),
                              preferred_element_type=jnp.float32)
    idx = _argmin_scan(x_sq, xc2, cbsq_ref[...], _K)
    quant = _exact_gather(cbpl_ref[...], idx, _K)
    quant_ref[...] = quant
    idx_ref[...] = idx[:, None]
    r_buf[i % 2] = x - quant


def kernel(x, codebook, residual_codebook):
    cbm2 = -2.0 * codebook
    rcbm2 = -2.0 * residual_codebook
    cbsq = jnp.sum(codebook * codebook, axis=1)[None, :]
    rcbsq = jnp.sum(residual_codebook * residual_codebook, axis=1)[None, :]
    cbpl = _byte_planes(codebook)
    rcbpl = _byte_planes(residual_codebook)

    nb = _N // _BN
    s1 = lambda i: (jnp.minimum(i, nb - 1), 0)
    s2 = lambda i: (jnp.maximum(i - 1, 0), 0)
    const = lambda i: (0, 0)
    out = pl.pallas_call(
        _rvq_body,
        grid=(nb + 1,),
        in_specs=[
            pl.BlockSpec((_BN, _C), s1),
            pl.BlockSpec((_K, _C), const),
            pl.BlockSpec((_KR, _C), const),
            pl.BlockSpec((1, _K), const),
            pl.BlockSpec((1, _KR), const),
            pl.BlockSpec((_K, 4 * _C), const),
            pl.BlockSpec((_KR, 4 * _C), const),
        ],
        out_specs=[
            pl.BlockSpec((_BN, _C), s1),
            pl.BlockSpec((_BN, 1), s1),
            pl.BlockSpec((_BN, _C), s2),
            pl.BlockSpec((_BN, 1), s2),
        ],
        out_shape=[
            jax.ShapeDtypeStruct((_N, _C), jnp.float32),
            jax.ShapeDtypeStruct((_N, 1), jnp.int32),
            jax.ShapeDtypeStruct((_N, _C), jnp.float32),
            jax.ShapeDtypeStruct((_N, 1), jnp.int32),
        ],
        scratch_shapes=[pltpu.VMEM((2, _BN, _C), jnp.float32)],
    )(x, cbm2, rcbm2, cbsq, rcbsq, cbpl, rcbpl)
    quant, idx, quant_r, idx_r = out
    return (quant, idx[:, 0], quant_r, idx_r[:, 0])
